# baseline (speedup 1.0000x reference)
import functools

import jax
import jax.numpy as jnp
from jax import lax
from jax.experimental import pallas as pl
from jax.experimental.pallas import tpu as pltpu
from jax.experimental.pallas import tpu_sc as plsc

NC = 2
NS = 16
NW = NC * NS

EPS = 1e-3
CHUNK = 128
HJ = 8

_GDN = lax.GatherDimensionNumbers(
    offset_dims=(), collapsed_slice_dims=(0,), start_index_map=(0,))


def _shuffle(x, idx):
    return lax.gather(x, idx[:, None], _GDN, (1,),
                      mode=lax.GatherScatterMode.PROMISE_IN_BOUNDS)


def _lane_sum(x, iota):
    for k in (8, 4, 2, 1):
        x = x + _shuffle(x, iota ^ k)
    return x


def _rsqrt_newton(v):
    i = lax.bitcast_convert_type(v, jnp.int32)
    i = jnp.int32(0x5F3759DF) - lax.shift_right_logical(i, 1)
    y = lax.bitcast_convert_type(i, jnp.float32)
    for _ in range(2):
        y = y * (1.5 - 0.5 * v * y * y)
    return y


def _sc_fused(table, ids_flat, tt_flat, type_pad, pos200, gamma, beta):
    n_rows = ids_flat.shape[0]
    H = table.shape[1]
    S = pos200.shape[0]
    per_w = n_rows // NW
    chunks_per_w = per_w // CHUNK
    mesh = plsc.VectorSubcoreMesh(core_axis_name="c", subcore_axis_name="s")

    @functools.partial(
        pl.kernel,
        out_type=jax.ShapeDtypeStruct((n_rows, H), jnp.float32),
        mesh=mesh,
        scratch_types=[
            pltpu.VMEM((S, H), jnp.float32),
            pltpu.VMEM((per_w,), jnp.int32),
            pltpu.VMEM((CHUNK, H), jnp.float32),
            pltpu.VMEM((CHUNK, H), jnp.float32),
            pltpu.VMEM((CHUNK, H), jnp.float32),
            pltpu.VMEM((CHUNK, H), jnp.float32),
            pltpu.VMEM((per_w,), jnp.int32),
            pltpu.VMEM((8, H), jnp.float32),
            pltpu.VMEM((H,), jnp.float32),
            pltpu.VMEM((H,), jnp.float32),
            pltpu.SemaphoreType.DMA,
            pltpu.SemaphoreType.DMA,
            pltpu.SemaphoreType.DMA,
            pltpu.SemaphoreType.DMA,
        ],
    )
    def k(table_hbm, ids_hbm, tt_hbm, type_hbm, pos_hbm, gamma_hbm,
          beta_hbm, out_hbm, aug, ttv, rows0, rows1, out0, out1, idxv,
          typev, gv, bv, g0, g1, s0, s1):
        wid = lax.axis_index("s") * NC + lax.axis_index("c")
        base = wid * per_w

        pltpu.sync_copy(ids_hbm.at[pl.ds(base, per_w)], idxv)
        pltpu.sync_copy(tt_hbm.at[pl.ds(base, per_w)], ttv)
        pltpu.sync_copy(pos_hbm, aug)
        pltpu.sync_copy(type_hbm, typev)
        pltpu.sync_copy(gamma_hbm, gv)
        pltpu.sync_copy(beta_hbm, bv)

        def aug_body(p, carry):
            for j in range(HJ):
                sl = pl.ds(16 * j, 16)
                aug[p, sl] += typev[0, sl]
            return carry
        lax.fori_loop(0, S, aug_body, 0)

        def fire_gather(c, buf, sem):
            return pltpu.async_copy(
                table_hbm.at[idxv.at[pl.ds(c * CHUNK, CHUNK)]], buf, sem)

        def wait_gather(c, buf, sem):
            pltpu.make_async_copy(
                table_hbm.at[idxv.at[pl.ds(c * CHUNK, CHUNK)]], buf,
                sem).wait()

        def fire_store(c, buf, sem):
            return pltpu.async_copy(
                buf, out_hbm.at[pl.ds((base + c * CHUNK), CHUNK)], sem)

        def wait_store(c, buf, sem):
            pltpu.make_async_copy(
                buf, out_hbm.at[pl.ds((base + c * CHUNK), CHUNK)],
                sem).wait()

        iota = lax.iota(jnp.int32, 16)

        def compute(c, rows, outb):
            def gb(g, carry):
                gs = tuple(gv[pl.ds(16 * j, 16)] for j in range(HJ))
                bs = tuple(bv[pl.ds(16 * j, 16)] for j in range(HJ))
                dv = tuple(typev[1, pl.ds(16 * j, 16)] -
                           typev[0, pl.ds(16 * j, 16)] for j in range(HJ))
                goff = c * CHUNK + 16 * g
                ttgf = ttv[pl.ds(goff, 16)].astype(jnp.float32)
                pg = lax.rem(base + goff, S)

                def row_pos(t16):
                    pr = pg + t16
                    return jnp.where(pr >= S, pr - S, pr)

                sv_acc = None
                qv_acc = None
                for t16 in range(16):
                    t = g * 16 + t16
                    pr = row_pos(t16)
                    ttf = _shuffle(ttgf, (iota & 0) + t16)
                    sv0 = None
                    qv0 = None
                    for j in range(HJ):
                        sl = pl.ds(16 * j, 16)
                        y = rows[t, sl] + (aug[pr, sl] + ttf * dv[j])
                        y2 = y * y
                        sv0 = y if sv0 is None else sv0 + y
                        qv0 = y2 if qv0 is None else qv0 + y2
                    sv0 = _lane_sum(sv0, iota)
                    qv0 = _lane_sum(qv0, iota)
                    m = iota == t16
                    sv_acc = sv0 if sv_acc is None else \
                        jnp.where(m, sv0, sv_acc)
                    qv_acc = qv0 if qv_acc is None else \
                        jnp.where(m, qv0, qv_acc)

                mean_v = sv_acc * (1.0 / H)
                var_v = qv_acc * (1.0 / H) - mean_v * mean_v
                rstd_v = _rsqrt_newton(var_v + EPS)
                shift_v = -mean_v * rstd_v

                for t16 in range(16):
                    t = g * 16 + t16
                    pr = row_pos(t16)
                    ttf = _shuffle(ttgf, (iota & 0) + t16)
                    rstd = _shuffle(rstd_v, (iota & 0) + t16)
                    shift = _shuffle(shift_v, (iota & 0) + t16)
                    for j in range(HJ):
                        sl = pl.ds(16 * j, 16)
                        y = rows[t, sl] + (aug[pr, sl] + ttf * dv[j])
                        outb[t, sl] = (y * rstd + shift) * gs[j] + bs[j]
                return carry

            lax.fori_loop(0, CHUNK // 16, gb, 0)

        fire_gather(0, rows0, g0)

        def step(kk, carry):
            c0 = 2 * kk
            c1 = c0 + 1
            fire_gather(c1, rows1, g1)
            wait_gather(c0, rows0, g0)

            @pl.when(kk > 0)
            def _():
                wait_store(c0 - 2, out0, s0)
            compute(c0, rows0, out0)
            fire_store(c0, out0, s0)

            @pl.when(kk < chunks_per_w // 2 - 1)
            def _():
                fire_gather(c0 + 2, rows0, g0)
            wait_gather(c1, rows1, g1)

            @pl.when(kk > 0)
            def _():
                wait_store(c1 - 2, out1, s1)
            compute(c1, rows1, out1)
            fire_store(c1, out1, s1)
            return carry

        lax.fori_loop(0, chunks_per_w // 2, step, 0)
        wait_store(chunks_per_w - 2, out0, s0)
        wait_store(chunks_per_w - 1, out1, s1)

    return k(table, ids_flat, tt_flat, type_pad, pos200, gamma, beta)


def kernel(input_ids, token_type_ids, word_emb, type_emb, pos_emb, gamma, beta):
    B, S = input_ids.shape
    H = word_emb.shape[1]
    n_rows = B * S
    ids_flat = input_ids.reshape(n_rows)
    tt_flat = token_type_ids.reshape(n_rows)
    type_pad = jnp.pad(type_emb, ((0, 6), (0, 0)))
    out = _sc_fused(word_emb, ids_flat, tt_flat, type_pad, pos_emb[:S],
                    gamma, beta)
    return out.reshape(B, S, H)


# speedup vs baseline: 4.4958x; 4.4958x over previous
import functools

import jax
import jax.numpy as jnp
from jax import lax
from jax.experimental import pallas as pl
from jax.experimental.pallas import tpu as pltpu
from jax.experimental.pallas import tpu_sc as plsc

NC = 2
NS = 16
NW = NC * NS

EPS = 1e-3
ROWS_PER_STREAM = 64
NSLICE = 4


def _sc_gather(table, idx3d, n_rows):
    H = table.shape[1]
    chunks_per_w = idx3d.shape[1]
    mesh = plsc.VectorSubcoreMesh(core_axis_name="c", subcore_axis_name="s")

    @functools.partial(
        pl.kernel,
        out_type=jax.ShapeDtypeStruct((n_rows, H), jnp.float32),
        mesh=mesh,
        scratch_types=[
            pltpu.VMEM((chunks_per_w, ROWS_PER_STREAM), jnp.int32),
            pltpu.VMEM((ROWS_PER_STREAM, H), jnp.float32),
            pltpu.SemaphoreType.DMA,
        ],
    )
    def k(table_hbm, idx_hbm, out_hbm, idx_v, rows_v, sem):
        wid = lax.axis_index("s") * NC + lax.axis_index("c")
        base = wid * chunks_per_w
        pltpu.sync_copy(idx_hbm.at[wid], idx_v)

        def body(i, carry):
            pltpu.async_copy(table_hbm.at[idx_v.at[i]], rows_v, sem).wait()
            row0 = (base + i) * ROWS_PER_STREAM
            pltpu.sync_copy(rows_v, out_hbm.at[pl.ds(row0, ROWS_PER_STREAM)])
            return carry

        lax.fori_loop(0, chunks_per_w, body, 0)

    return k(table, idx3d)


def _tc_body(g_ref, tt_ref, type_ref, pos_ref, gamma_ref, beta_ref, o_ref):
    x = g_ref[...]
    tt = tt_ref[...].astype(jnp.float32)[..., None]
    t0 = type_ref[0]
    t1 = type_ref[1]
    x = x + t0 + tt * (t1 - t0) + pos_ref[...][None]
    mean = jnp.mean(x, axis=-1, keepdims=True)
    xc = x - mean
    var = jnp.mean(xc * xc, axis=-1, keepdims=True)
    y = xc * lax.rsqrt(var + EPS)
    o_ref[...] = y * gamma_ref[...] + beta_ref[...]


def _tc_add_ln(gathered, token_type_ids, type_emb, pos_slice, gamma, beta):
    B, S = token_type_ids.shape
    H = type_emb.shape[1]
    BB = 8
    grid = (B // BB,)
    return pl.pallas_call(
        _tc_body,
        grid=grid,
        in_specs=[
            pl.BlockSpec((BB, S, H), lambda i: (i, 0, 0)),
            pl.BlockSpec((BB, S), lambda i: (i, 0)),
            pl.BlockSpec((2, H), lambda i: (0, 0)),
            pl.BlockSpec((S, H), lambda i: (0, 0)),
            pl.BlockSpec((1, H), lambda i: (0, 0)),
            pl.BlockSpec((1, H), lambda i: (0, 0)),
        ],
        out_specs=pl.BlockSpec((BB, S, H), lambda i: (i, 0, 0)),
        out_shape=jax.ShapeDtypeStruct((B, S, H), jnp.float32),
        compiler_params=pltpu.CompilerParams(
            dimension_semantics=("arbitrary",)),
    )(gathered, token_type_ids, type_emb, pos_slice, gamma, beta)


def kernel(input_ids, token_type_ids, word_emb, type_emb, pos_emb, gamma, beta):
    B, S = input_ids.shape
    H = word_emb.shape[1]
    bs = B // NSLICE
    n_rows = bs * S
    outs = []
    for i in range(NSLICE):
        ids_i = lax.slice_in_dim(input_ids, i * bs, (i + 1) * bs)
        tt_i = lax.slice_in_dim(token_type_ids, i * bs, (i + 1) * bs)
        idx3d = ids_i.reshape(NW, n_rows // (NW * ROWS_PER_STREAM),
                              ROWS_PER_STREAM)
        g_i = _sc_gather(word_emb, idx3d, n_rows).reshape(bs, S, H)
        outs.append(_tc_add_ln(g_i, tt_i, type_emb, pos_emb[:S],
                               gamma.reshape(1, H), beta.reshape(1, H)))
    return jnp.concatenate(outs, axis=0)


# speedup vs baseline: 5.7699x; 1.2834x over previous
import functools

import jax
import jax.numpy as jnp
from jax import lax
from jax.experimental import pallas as pl
from jax.experimental.pallas import tpu as pltpu
from jax.experimental.pallas import tpu_sc as plsc

NC = 2
NS = 16
NW = NC * NS

EPS = 1e-3
ROWS_PER_STREAM = 64
NSLICE = 4


def _sc_gather(table, idx3d, n_rows):
    H = table.shape[1]
    chunks_per_w = idx3d.shape[1]
    mesh = plsc.VectorSubcoreMesh(core_axis_name="c", subcore_axis_name="s")

    @functools.partial(
        pl.kernel,
        out_type=jax.ShapeDtypeStruct((n_rows, H), jnp.float32),
        mesh=mesh,
        scratch_types=[
            pltpu.VMEM((chunks_per_w, ROWS_PER_STREAM), jnp.int32),
            pltpu.VMEM((ROWS_PER_STREAM, H), jnp.float32),
            pltpu.SemaphoreType.DMA,
        ],
    )
    def k(table_hbm, idx_hbm, out_hbm, idx_v, rows_v, sem):
        wid = lax.axis_index("s") * NC + lax.axis_index("c")
        base = wid * chunks_per_w
        pltpu.sync_copy(idx_hbm.at[wid], idx_v)

        def body(i, carry):
            pltpu.async_copy(table_hbm.at[idx_v.at[i]], rows_v, sem).wait()
            row0 = (base + i) * ROWS_PER_STREAM
            pltpu.sync_copy(rows_v, out_hbm.at[pl.ds(row0, ROWS_PER_STREAM)])
            return carry

        lax.fori_loop(0, chunks_per_w, body, 0)

    return k(table, idx3d)


def _tc_body_first(g_ref, tt_ref, type_ref, pos_ref, gamma_ref, beta_ref,
                   o_ref):
    x = g_ref[...]
    tt = tt_ref[...].astype(jnp.float32)[..., None]
    t0 = type_ref[0]
    t1 = type_ref[1]
    x = x + t0 + tt * (t1 - t0) + pos_ref[...][None]
    mean = jnp.mean(x, axis=-1, keepdims=True)
    xc = x - mean
    var = jnp.mean(xc * xc, axis=-1, keepdims=True)
    y = xc * lax.rsqrt(var + EPS)
    o_ref[...] = y * gamma_ref[...] + beta_ref[...]


def _tc_body_acc(acc_ref, g_ref, tt_ref, type_ref, pos_ref, gamma_ref,
                 beta_ref, o_ref):
    del acc_ref
    _tc_body_first(g_ref, tt_ref, type_ref, pos_ref, gamma_ref, beta_ref,
                   o_ref)


BB = 16


def _tc_add_ln(acc, off_blocks, gathered, token_type_ids, type_emb,
               pos_slice, gamma, beta, full_b):
    bs, S = token_type_ids.shape
    H = type_emb.shape[1]
    grid = (bs // BB,)
    data_specs = [
        pl.BlockSpec((BB, S, H), lambda i: (i, 0, 0)),
        pl.BlockSpec((BB, S), lambda i: (i, 0)),
        pl.BlockSpec((2, H), lambda i: (0, 0)),
        pl.BlockSpec((S, H), lambda i: (0, 0)),
        pl.BlockSpec((1, H), lambda i: (0, 0)),
        pl.BlockSpec((1, H), lambda i: (0, 0)),
    ]
    out_spec = pl.BlockSpec((BB, S, H), lambda i: (off_blocks + i, 0, 0))
    out_shape = jax.ShapeDtypeStruct((full_b, S, H), jnp.float32)
    params = pltpu.CompilerParams(dimension_semantics=("arbitrary",))
    if acc is None:
        return pl.pallas_call(
            _tc_body_first, grid=grid, in_specs=data_specs,
            out_specs=out_spec, out_shape=out_shape,
            compiler_params=params,
        )(gathered, token_type_ids, type_emb, pos_slice, gamma, beta)
    return pl.pallas_call(
        _tc_body_acc, grid=grid,
        in_specs=[pl.BlockSpec(memory_space=pl.ANY)] + data_specs,
        out_specs=out_spec, out_shape=out_shape,
        input_output_aliases={0: 0},
        compiler_params=params,
    )(acc, gathered, token_type_ids, type_emb, pos_slice, gamma, beta)


def kernel(input_ids, token_type_ids, word_emb, type_emb, pos_emb, gamma, beta):
    B, S = input_ids.shape
    H = word_emb.shape[1]
    bs = B // NSLICE
    n_rows = bs * S
    acc = None
    for i in range(NSLICE):
        ids_i = lax.slice_in_dim(input_ids, i * bs, (i + 1) * bs)
        tt_i = lax.slice_in_dim(token_type_ids, i * bs, (i + 1) * bs)
        idx3d = ids_i.reshape(NW, n_rows // (NW * ROWS_PER_STREAM),
                              ROWS_PER_STREAM)
        g_i = _sc_gather(word_emb, idx3d, n_rows).reshape(bs, S, H)
        acc = _tc_add_ln(acc, i * (bs // BB), g_i, tt_i, type_emb,
                         pos_emb[:S], gamma.reshape(1, H),
                         beta.reshape(1, H), B)
    return acc


# speedup vs baseline: 6.5775x; 1.1400x over previous
import functools

import jax
import jax.numpy as jnp
from jax import lax
from jax.experimental import pallas as pl
from jax.experimental.pallas import tpu as pltpu
from jax.experimental.pallas import tpu_sc as plsc

NC = 2
NS = 16
NW = NC * NS

EPS = 1e-3
ROWS_PER_STREAM = 128
NSLICE = 2


def _sc_gather(table, idx3d, n_rows):
    H = table.shape[1]
    chunks_per_w = idx3d.shape[1]
    mesh = plsc.VectorSubcoreMesh(core_axis_name="c", subcore_axis_name="s")

    @functools.partial(
        pl.kernel,
        out_type=jax.ShapeDtypeStruct((n_rows, H), jnp.float32),
        mesh=mesh,
        scratch_types=[
            pltpu.VMEM((chunks_per_w, ROWS_PER_STREAM), jnp.int32),
            pltpu.VMEM((ROWS_PER_STREAM, H), jnp.float32),
            pltpu.SemaphoreType.DMA,
        ],
    )
    def k(table_hbm, idx_hbm, out_hbm, idx_v, rows_v, sem):
        wid = lax.axis_index("s") * NC + lax.axis_index("c")
        base = wid * chunks_per_w
        pltpu.sync_copy(idx_hbm.at[wid], idx_v)

        def body(i, carry):
            pltpu.async_copy(table_hbm.at[idx_v.at[i]], rows_v, sem).wait()
            row0 = (base + i) * ROWS_PER_STREAM
            pltpu.sync_copy(rows_v, out_hbm.at[pl.ds(row0, ROWS_PER_STREAM)])
            return carry

        lax.fori_loop(0, chunks_per_w, body, 0)

    return k(table, idx3d)


def _tc_body_first(g_ref, tt_ref, type_ref, pos_ref, gamma_ref, beta_ref,
                   o_ref):
    x = g_ref[...]
    tt = tt_ref[...].astype(jnp.float32)[..., None]
    t0 = type_ref[0]
    t1 = type_ref[1]
    x = x + t0 + tt * (t1 - t0) + pos_ref[...][None]
    mean = jnp.mean(x, axis=-1, keepdims=True)
    xc = x - mean
    var = jnp.mean(xc * xc, axis=-1, keepdims=True)
    y = xc * lax.rsqrt(var + EPS)
    o_ref[...] = y * gamma_ref[...] + beta_ref[...]


def _tc_body_acc(acc_ref, g_ref, tt_ref, type_ref, pos_ref, gamma_ref,
                 beta_ref, o_ref):
    del acc_ref
    _tc_body_first(g_ref, tt_ref, type_ref, pos_ref, gamma_ref, beta_ref,
                   o_ref)


BB = 16


def _tc_add_ln(acc, off_blocks, gathered, token_type_ids, type_emb,
               pos_slice, gamma, beta, full_b):
    bs, S = token_type_ids.shape
    H = type_emb.shape[1]
    grid = (bs // BB,)
    data_specs = [
        pl.BlockSpec((BB, S, H), lambda i: (i, 0, 0)),
        pl.BlockSpec((BB, S), lambda i: (i, 0)),
        pl.BlockSpec((2, H), lambda i: (0, 0)),
        pl.BlockSpec((S, H), lambda i: (0, 0)),
        pl.BlockSpec((1, H), lambda i: (0, 0)),
        pl.BlockSpec((1, H), lambda i: (0, 0)),
    ]
    out_spec = pl.BlockSpec((BB, S, H), lambda i: (off_blocks + i, 0, 0))
    out_shape = jax.ShapeDtypeStruct((full_b, S, H), jnp.float32)
    params = pltpu.CompilerParams(dimension_semantics=("arbitrary",))
    if acc is None:
        return pl.pallas_call(
            _tc_body_first, grid=grid, in_specs=data_specs,
            out_specs=out_spec, out_shape=out_shape,
            compiler_params=params,
        )(gathered, token_type_ids, type_emb, pos_slice, gamma, beta)
    return pl.pallas_call(
        _tc_body_acc, grid=grid,
        in_specs=[pl.BlockSpec(memory_space=pl.ANY)] + data_specs,
        out_specs=out_spec, out_shape=out_shape,
        input_output_aliases={0: 0},
        compiler_params=params,
    )(acc, gathered, token_type_ids, type_emb, pos_slice, gamma, beta)


def kernel(input_ids, token_type_ids, word_emb, type_emb, pos_emb, gamma, beta):
    B, S = input_ids.shape
    H = word_emb.shape[1]
    bs = B // NSLICE
    n_rows = bs * S
    acc = None
    for i in range(NSLICE):
        ids_i = lax.slice_in_dim(input_ids, i * bs, (i + 1) * bs)
        tt_i = lax.slice_in_dim(token_type_ids, i * bs, (i + 1) * bs)
        idx3d = ids_i.reshape(NW, n_rows // (NW * ROWS_PER_STREAM),
                              ROWS_PER_STREAM)
        g_i = _sc_gather(word_emb, idx3d, n_rows).reshape(bs, S, H)
        acc = _tc_add_ln(acc, i * (bs // BB), g_i, tt_i, type_emb,
                         pos_emb[:S], gamma.reshape(1, H),
                         beta.reshape(1, H), B)
    return acc
